# baseline (speedup 1.0000x reference)
import functools

import jax
import jax.numpy as jnp
from jax import lax
from jax.experimental import pallas as pl
from jax.experimental.pallas import tpu as pltpu
from jax.experimental.pallas import tpu_sc as plsc

VOCAB = 1000000
EMBED = 64
BATCH = 16384
X_MAX = 100.0
ALPHA = 0.75

NC = 2
NS = 16
NW = NC * NS
BPW = BATCH // NW
CHUNK = 128
NCHUNK = BPW // CHUNK
L = 16
CPACK = 2 * EMBED
NPASS = 2
EPP = BPW // NPASS
CPP = NCHUNK // NPASS

_LN2 = 0.6931471805599453
_LN_XMAX = 4.605170185988092
_SQRT2 = 1.4142135623730951

_MESH = plsc.VectorSubcoreMesh(
    core_axis_name="c", subcore_axis_name="s", num_cores=NC, num_subcores=NS
)
_PARAMS = pltpu.CompilerParams(use_tc_tiling_on_sc=False)


def _vlog(x):
    bits = lax.bitcast_convert_type(x, jnp.int32)
    e = (bits >> 23) - 127
    m = lax.bitcast_convert_type((bits & 0x007FFFFF) | 0x3F800000, jnp.float32)
    big = m > _SQRT2
    e = jnp.where(big, e + 1, e)
    m = jnp.where(big, m * 0.5, m)
    t = (m - 1.0) / (m + 1.0)
    t2 = t * t
    poly = 2.0 * t * (1.0 + t2 * (1.0 / 3.0 + t2 * (0.2 + t2 * (1.0 / 7.0))))
    return e.astype(jnp.float32) * _LN2 + poly


@functools.partial(
    pl.kernel,
    out_type=(
        jax.ShapeDtypeStruct((BATCH, CPACK), jnp.float32),
        jax.ShapeDtypeStruct((BATCH,), jnp.float32),
    ),
    mesh=_MESH,
    compiler_params=_PARAMS,
    scratch_types=[
        pltpu.VMEM((NCHUNK, CHUNK), jnp.int32),
        pltpu.VMEM((NCHUNK, CHUNK), jnp.int32),
        pltpu.VMEM((BPW, CPACK), jnp.float32),
        pltpu.VMEM((BPW,), jnp.float32),
        pltpu.SemaphoreType.DMA,
    ],
)
def _k1_focal(femb2, fbias, fidx, rows_out, fb_out, idx_v, idx2_v, rows, fb_v, sem):
    wid = lax.axis_index("s") * NC + lax.axis_index("c")
    base = wid * BPW
    for i in range(NCHUNK):
        pltpu.sync_copy(fidx.at[pl.ds(base + i * CHUNK, CHUNK)], idx_v.at[i])

    def tf_body(i, carry):
        def inner(k, carry2):
            sl = pl.ds(k * L, L)
            idx2_v.at[i][sl] = idx_v.at[i][sl] >> 1
            return carry2
        return lax.fori_loop(0, CHUNK // L, inner, carry)

    lax.fori_loop(0, NCHUNK, tf_body, 0)

    copies = []
    for i in range(NCHUNK):
        sl = pl.ds(i * CHUNK, CHUNK)
        copies.append(pltpu.async_copy(femb2.at[idx2_v.at[i]], rows.at[sl], sem))
        copies.append(pltpu.async_copy(fbias.at[idx_v.at[i]], fb_v.at[sl], sem))
    for c in copies:
        c.wait()
    pltpu.sync_copy(rows, rows_out.at[pl.ds(base, BPW)])
    pltpu.sync_copy(fb_v, fb_out.at[pl.ds(base, BPW)])


def _k2_body(cemb2, cbias, cnt, fidx, cidx, frows_hbm, fb_hbm, out_hbm,
             idxf_v, idxc_v, idxc2_v, hf_v, hc_v, frows, crows,
             fb_v, cb_v, cnt_v, w_v, lc_v, out_v, sem):
    wid = lax.axis_index("s") * NC + lax.axis_index("c")
    base = wid * BPW

    for i in range(NCHUNK):
        pltpu.sync_copy(fidx.at[pl.ds(base + i * CHUNK, CHUNK)], idxf_v.at[i])
        pltpu.sync_copy(cidx.at[pl.ds(base + i * CHUNK, CHUNK)], idxc_v.at[i])
    pltpu.sync_copy(cnt.at[pl.ds(base, BPW)], cnt_v)
    pltpu.sync_copy(fb_hbm.at[pl.ds(base, BPW)], fb_v)

    def tf_body(i, carry):
        def inner(k, carry2):
            sl = pl.ds(k * L, L)
            gsl = pl.ds(i * CHUNK + k * L, L)
            hf_v[gsl] = (idxf_v.at[i][sl] & 1) * EMBED
            vc = idxc_v.at[i][sl]
            idxc2_v.at[i][sl] = vc >> 1
            hc_v[gsl] = (vc & 1) * EMBED
            return carry2
        return lax.fori_loop(0, CHUNK // L, inner, carry)

    lax.fori_loop(0, NCHUNK, tf_body, 0)

    bias_copies = []
    for i in range(NCHUNK):
        sl = pl.ds(i * CHUNK, CHUNK)
        bias_copies.append(pltpu.async_copy(cbias.at[idxc_v.at[i]], cb_v.at[sl], sem))

    def fire_pass(p):
        cps = []
        for q in range(CPP):
            i = p * CPP + q
            sl = pl.ds(q * CHUNK, CHUNK)
            cps.append(pltpu.async_copy(cemb2.at[idxc2_v.at[i]], crows.at[sl], sem))
            cps.append(pltpu.async_copy(
                frows_hbm.at[pl.ds(base + i * CHUNK, CHUNK)], frows.at[sl], sem))
        return cps

    pass_copies = fire_pass(0)

    def wl_body(g, carry):
        sl = pl.ds(g * L, L)
        c = cnt_v[sl]
        lc = _vlog(c)
        w = jnp.minimum(jnp.exp(ALPHA * (lc - _LN_XMAX)), 1.0)
        lc_v[sl] = lc
        w_v[sl] = w
        return carry

    lax.fori_loop(0, BPW // L, wl_body, 0)

    for c in bias_copies:
        c.wait()

    lanes = lax.iota(jnp.int32, L)
    perms = [lanes ^ sh for sh in (1, 2, 4, 8)]

    lossvec = jnp.zeros((L,), jnp.float32)
    for p in range(NPASS):
        for c in pass_copies:
            c.wait()
        if p + 1 < NPASS:
            next_copies = fire_pass(p + 1)

        def group_body(g, lv, _p=p):
            gsl = pl.ds((_p * EPP // L + g) * L, L)
            s16 = fb_v[gsl] + cb_v[gsl] + lc_v[gsl]
            w16 = w_v[gsl]
            hf16 = hf_v[gsl]
            hc16 = hc_v[gsl]
            d_vec = jnp.zeros((L,), jnp.float32)
            for k in range(L):
                b = g * L + k
                fr = frows.at[b]
                cr = crows.at[b]
                offf = hf16[k]
                offc = hc16[k]
                pv = fr[pl.ds(offf, L)] * cr[pl.ds(offc, L)]
                for j in range(1, EMBED // L):
                    pv = pv + fr[pl.ds(offf + j * L, L)] * cr[pl.ds(offc + j * L, L)]
                for perm in perms:
                    pv = pv + jnp.take(pv, perm)
                d_vec = jnp.where(lanes == k, pv, d_vec)
            expr = d_vec + s16
            return lv + w16 * (expr * expr)

        lossvec = lax.fori_loop(0, EPP // L, group_body, lossvec)
        if p + 1 < NPASS:
            pass_copies = next_copies

    out_v[...] = lossvec
    pltpu.sync_copy(out_v, out_hbm.at[wid])


@functools.partial(
    pl.kernel,
    out_type=jax.ShapeDtypeStruct((NW, L), jnp.float32),
    mesh=_MESH,
    compiler_params=_PARAMS,
    scratch_types=[
        pltpu.VMEM((NCHUNK, CHUNK), jnp.int32),
        pltpu.VMEM((NCHUNK, CHUNK), jnp.int32),
        pltpu.VMEM((NCHUNK, CHUNK), jnp.int32),
        pltpu.VMEM((BPW,), jnp.int32),
        pltpu.VMEM((BPW,), jnp.int32),
        pltpu.VMEM((EPP, CPACK), jnp.float32),
        pltpu.VMEM((EPP, CPACK), jnp.float32),
        pltpu.VMEM((BPW,), jnp.float32),
        pltpu.VMEM((BPW,), jnp.float32),
        pltpu.VMEM((BPW,), jnp.float32),
        pltpu.VMEM((BPW,), jnp.float32),
        pltpu.VMEM((BPW,), jnp.float32),
        pltpu.VMEM((L,), jnp.float32),
        pltpu.SemaphoreType.DMA,
    ],
)
def _k2_context(cemb2, cbias, cnt, fidx, cidx, frows_hbm, fb_hbm, out_hbm, *scratch):
    _k2_body(cemb2, cbias, cnt, fidx, cidx, frows_hbm, fb_hbm, out_hbm, *scratch)


def kernel(focal_embeddings, context_embeddings, focal_biases, context_biases,
           coocurrence_count, focal_input, context_input):
    femb2 = focal_embeddings.reshape(VOCAB // 2, CPACK)
    cemb2 = context_embeddings.reshape(VOCAB // 2, CPACK)
    fidx = focal_input.astype(jnp.int32)
    cidx = context_input.astype(jnp.int32)
    frows, fb = _k1_focal(femb2, focal_biases, fidx)
    partials = _k2_context(cemb2, context_biases, coocurrence_count,
                           fidx, cidx, frows, fb)
    return jnp.sum(partials)
